# Initial kernel scaffold; baseline (speedup 1.0000x reference)
import jax
import jax.numpy as jnp
from jax.experimental import pallas as pl


def kernel(x, edge_index, synapse_weights, neuron_biases):
    raise NotImplementedError("write your pallas kernel here")


# speedup vs baseline: 252.9124x; 252.9124x over previous
import functools

import jax
import jax.numpy as jnp
from jax import lax
from jax.experimental import pallas as pl
from jax.experimental.pallas import tpu as pltpu
from jax.experimental.pallas import tpu_sc as plsc

N = 100000
E = 6400000
NC = 2
NS = 16
NW = NC * NS
EPW = E // NW
C = 10000
NCHUNK = EPW // C
SLICE = 6272
FS = 3136

_mesh = plsc.VectorSubcoreMesh(core_axis_name="c", subcore_axis_name="s")


def _tanh16(t):
    e = jnp.exp(2.0 * t)
    return 1.0 - 2.0 / (e + 1.0)


def _make_step(first: bool):
    scratch = [
        pltpu.VMEM_SHARED((N,), jnp.float32),
        pltpu.VMEM_SHARED((N,), jnp.float32),
        pltpu.VMEM((SLICE,), jnp.float32),
        pltpu.VMEM((SLICE,), jnp.float32),
        pltpu.VMEM((SLICE,), jnp.float32),
        pltpu.VMEM((C,), jnp.int32),
        pltpu.VMEM((C,), jnp.int32),
        pltpu.VMEM((C,), jnp.float32),
        pltpu.VMEM((C,), jnp.float32),
        pltpu.SemaphoreType.DMA,
    ]

    @functools.partial(
        pl.kernel,
        out_type=jax.ShapeDtypeStruct((NC * N,), jnp.float32),
        mesh=_mesh,
        scratch_types=scratch,
    )
    def step(vin_hbm, edges_hbm, w_hbm, b_hbm, out_hbm,
             v_sh, acc_sh, stage, tmp_a, tmp_b, src_v, dst_v, w_v, msg_v, sem):
        cid = lax.axis_index("c")
        sid = lax.axis_index("s")
        wid = sid * NC + cid
        off = jnp.minimum(sid * SLICE, N - SLICE)

        if first:
            pltpu.sync_copy(vin_hbm.at[pl.ds(off, SLICE)], stage)
        else:
            pltpu.sync_copy(vin_hbm.at[pl.ds(off, SLICE)], stage)
            pltpu.sync_copy(vin_hbm.at[pl.ds(N + off, SLICE)], tmp_a)
            pltpu.sync_copy(b_hbm.at[pl.ds(off, SLICE)], tmp_b)

            def combine_body(i, _):
                s = pl.ds(i * 16, 16)
                stage[s] = _tanh16(stage[s] + tmp_a[s] + tmp_b[s])
                return 0
            lax.fori_loop(0, SLICE // 16, combine_body, 0)

        def zero_body(i, _):
            tmp_a[pl.ds(i * 16, 16)] = jnp.zeros((16,), jnp.float32)
            return 0
        lax.fori_loop(0, SLICE // 16, zero_body, 0)

        pltpu.sync_copy(stage, v_sh.at[pl.ds(off, SLICE)])
        pltpu.sync_copy(tmp_a, acc_sh.at[pl.ds(off, SLICE)])
        plsc.subcore_barrier()

        ebase = wid * EPW

        def chunk_body(k, _):
            b0 = ebase + k * C
            pltpu.sync_copy(edges_hbm.at[pl.ds(b0, C)], src_v)
            pltpu.sync_copy(edges_hbm.at[pl.ds(E + b0, C)], dst_v)
            pltpu.sync_copy(w_hbm.at[pl.ds(b0, C)], w_v)
            pltpu.async_copy(v_sh.at[src_v], msg_v, sem).wait()

            def mul_body(i, _):
                s = pl.ds(i * 16, 16)
                msg_v[s] = msg_v[s] * w_v[s]
                return 0
            lax.fori_loop(0, C // 16, mul_body, 0)

            pltpu.sync_copy(msg_v, acc_sh.at[dst_v], add=True)
            return 0
        lax.fori_loop(0, NCHUNK, chunk_body, 0)
        plsc.subcore_barrier()

        pltpu.sync_copy(acc_sh.at[pl.ds(off, SLICE)], stage)
        pltpu.sync_copy(stage, out_hbm.at[pl.ds(cid * N + off, SLICE)])

    return step


_step_first = _make_step(first=True)
_step_next = _make_step(first=False)


@functools.partial(
    pl.kernel,
    out_type=jax.ShapeDtypeStruct((N,), jnp.float32),
    mesh=_mesh,
    scratch_types=[
        pltpu.VMEM((FS,), jnp.float32),
        pltpu.VMEM((FS,), jnp.float32),
        pltpu.VMEM((FS,), jnp.float32),
    ],
)
def _final(p_hbm, b_hbm, out_hbm, p0b, p1b, bb):
    cid = lax.axis_index("c")
    sid = lax.axis_index("s")
    wid = sid * NC + cid
    off = jnp.minimum(wid * FS, N - FS)
    pltpu.sync_copy(p_hbm.at[pl.ds(off, FS)], p0b)
    pltpu.sync_copy(p_hbm.at[pl.ds(N + off, FS)], p1b)
    pltpu.sync_copy(b_hbm.at[pl.ds(off, FS)], bb)

    def body(i, _):
        s = pl.ds(i * 16, 16)
        p0b[s] = _tanh16(p0b[s] + p1b[s] + bb[s])
        return 0
    lax.fori_loop(0, FS // 16, body, 0)
    pltpu.sync_copy(p0b, out_hbm.at[pl.ds(off, FS)])


def kernel(x, edge_index, synapse_weights, neuron_biases):
    edges = edge_index.reshape(-1)
    p = _step_first(x, edges, synapse_weights, neuron_biases)
    p = _step_next(p, edges, synapse_weights, neuron_biases)
    p = _step_next(p, edges, synapse_weights, neuron_biases)
    return _final(p, neuron_biases)


# speedup vs baseline: 351.9477x; 1.3916x over previous
import functools

import jax
import jax.numpy as jnp
from jax import lax
from jax.experimental import pallas as pl
from jax.experimental.pallas import tpu as pltpu
from jax.experimental.pallas import tpu_sc as plsc

N = 100000
E = 6400000
NC = 2
NS = 16
NW = NC * NS
EPW = E // NW
C = 10000
NCHUNK = EPW // C
SLICE = 6272
FS = 3136

_mesh = plsc.VectorSubcoreMesh(core_axis_name="c", subcore_axis_name="s")


def _tanh16(t):
    e = jnp.exp(2.0 * t)
    return 1.0 - 2.0 / (e + 1.0)


def _make_step(first: bool):
    scratch = [
        pltpu.VMEM_SHARED((N,), jnp.float32),
        pltpu.VMEM_SHARED((N,), jnp.float32),
        pltpu.VMEM((SLICE,), jnp.float32),
        pltpu.VMEM((SLICE,), jnp.float32),
        pltpu.VMEM((SLICE,), jnp.float32),
        pltpu.VMEM((C,), jnp.int32),
        pltpu.VMEM((C,), jnp.int32),
        pltpu.VMEM((C,), jnp.float32),
        pltpu.VMEM((C,), jnp.int32),
        pltpu.VMEM((C,), jnp.int32),
        pltpu.VMEM((C,), jnp.float32),
        pltpu.VMEM((C,), jnp.float32),
        pltpu.SemaphoreType.DMA,
        pltpu.SemaphoreType.DMA,
    ]

    @functools.partial(
        pl.kernel,
        out_type=jax.ShapeDtypeStruct((NC * N,), jnp.float32),
        mesh=_mesh,
        scratch_types=scratch,
    )
    def step(vin_hbm, edges_hbm, w_hbm, b_hbm, out_hbm,
             v_sh, acc_sh, stage, tmp_a, tmp_b,
             src_a, dst_a, w_a, src_b, dst_b, w_b, msg_v, gsem, lsem):
        cid = lax.axis_index("c")
        sid = lax.axis_index("s")
        wid = sid * NC + cid
        off = jnp.minimum(sid * SLICE, N - SLICE)
        ebase = wid * EPW

        def issue_loads(g, sb, db, wb):
            b0 = ebase + g * C
            pltpu.async_copy(edges_hbm.at[pl.ds(b0, C)], sb, lsem)
            pltpu.async_copy(edges_hbm.at[pl.ds(E + b0, C)], db, lsem)
            pltpu.async_copy(w_hbm.at[pl.ds(b0, C)], wb, lsem)

        def wait_loads(g, sb, db, wb):
            b0 = ebase + g * C
            pltpu.make_async_copy(edges_hbm.at[pl.ds(b0, C)], sb, lsem).wait()
            pltpu.make_async_copy(edges_hbm.at[pl.ds(E + b0, C)], db, lsem).wait()
            pltpu.make_async_copy(w_hbm.at[pl.ds(b0, C)], wb, lsem).wait()

        issue_loads(0, src_a, dst_a, w_a)

        if first:
            pltpu.sync_copy(vin_hbm.at[pl.ds(off, SLICE)], stage)
        else:
            pltpu.sync_copy(vin_hbm.at[pl.ds(off, SLICE)], stage)
            pltpu.sync_copy(vin_hbm.at[pl.ds(N + off, SLICE)], tmp_a)
            pltpu.sync_copy(b_hbm.at[pl.ds(off, SLICE)], tmp_b)

            def combine_body(i, _):
                s = pl.ds(i * 16, 16)
                stage[s] = _tanh16(stage[s] + tmp_a[s] + tmp_b[s])
                return 0
            lax.fori_loop(0, SLICE // 16, combine_body, 0)

        def zero_body(i, _):
            tmp_a[pl.ds(i * 16, 16)] = jnp.zeros((16,), jnp.float32)
            return 0
        lax.fori_loop(0, SLICE // 16, zero_body, 0)

        pltpu.sync_copy(stage, v_sh.at[pl.ds(off, SLICE)])
        pltpu.sync_copy(tmp_a, acc_sh.at[pl.ds(off, SLICE)])
        plsc.subcore_barrier()

        def process(sb, db, wb):
            pltpu.async_copy(v_sh.at[sb], msg_v, gsem).wait()

            @plsc.parallel_loop(0, C // 16, unroll=8)
            def _mul(i):
                s = pl.ds(i * 16, 16)
                msg_v[s] = msg_v[s] * wb[s]

            pltpu.sync_copy(msg_v, acc_sh.at[db], add=True)

        def pair_body(h, _):
            g0 = 2 * h
            wait_loads(g0, src_a, dst_a, w_a)
            issue_loads(g0 + 1, src_b, dst_b, w_b)
            process(src_a, dst_a, w_a)
            wait_loads(g0 + 1, src_b, dst_b, w_b)

            @pl.when(g0 + 2 < NCHUNK)
            def _():
                issue_loads(g0 + 2, src_a, dst_a, w_a)

            process(src_b, dst_b, w_b)
            return 0
        lax.fori_loop(0, NCHUNK // 2, pair_body, 0)
        plsc.subcore_barrier()

        pltpu.sync_copy(acc_sh.at[pl.ds(off, SLICE)], stage)
        pltpu.sync_copy(stage, out_hbm.at[pl.ds(cid * N + off, SLICE)])

    return step


_step_first = _make_step(first=True)
_step_next = _make_step(first=False)


@functools.partial(
    pl.kernel,
    out_type=jax.ShapeDtypeStruct((N,), jnp.float32),
    mesh=_mesh,
    scratch_types=[
        pltpu.VMEM((FS,), jnp.float32),
        pltpu.VMEM((FS,), jnp.float32),
        pltpu.VMEM((FS,), jnp.float32),
    ],
)
def _final(p_hbm, b_hbm, out_hbm, p0b, p1b, bb):
    cid = lax.axis_index("c")
    sid = lax.axis_index("s")
    wid = sid * NC + cid
    off = jnp.minimum(wid * FS, N - FS)
    pltpu.sync_copy(p_hbm.at[pl.ds(off, FS)], p0b)
    pltpu.sync_copy(p_hbm.at[pl.ds(N + off, FS)], p1b)
    pltpu.sync_copy(b_hbm.at[pl.ds(off, FS)], bb)

    def body(i, _):
        s = pl.ds(i * 16, 16)
        p0b[s] = _tanh16(p0b[s] + p1b[s] + bb[s])
        return 0
    lax.fori_loop(0, FS // 16, body, 0)
    pltpu.sync_copy(p0b, out_hbm.at[pl.ds(off, FS)])


def kernel(x, edge_index, synapse_weights, neuron_biases):
    edges = edge_index.reshape(-1)
    p = _step_first(x, edges, synapse_weights, neuron_biases)
    p = _step_next(p, edges, synapse_weights, neuron_biases)
    p = _step_next(p, edges, synapse_weights, neuron_biases)
    return _final(p, neuron_biases)


# speedup vs baseline: 387.7251x; 1.1017x over previous
import functools

import jax
import jax.numpy as jnp
from jax import lax
from jax.experimental import pallas as pl
from jax.experimental.pallas import tpu as pltpu
from jax.experimental.pallas import tpu_sc as plsc

N = 100000
E = 6400000
NC = 2
NS = 16
NW = NC * NS
EPW = E // NW
C = 4000
NCHUNK = EPW // C
SLICE = 6272
FS = 3136

_mesh = plsc.VectorSubcoreMesh(core_axis_name="c", subcore_axis_name="s")


def _tanh16(t):
    e = jnp.exp(2.0 * t)
    return 1.0 - 2.0 / (e + 1.0)


def _make_step(first: bool):
    scratch = [
        pltpu.VMEM_SHARED((N,), jnp.float32),
        pltpu.VMEM_SHARED((N,), jnp.float32),
        pltpu.VMEM((SLICE,), jnp.float32),
        pltpu.VMEM((SLICE,), jnp.float32),
        pltpu.VMEM((SLICE,), jnp.float32),
        pltpu.VMEM((C,), jnp.int32),
        pltpu.VMEM((C,), jnp.int32),
        pltpu.VMEM((C,), jnp.int32),
        pltpu.VMEM((C,), jnp.int32),
        pltpu.VMEM((C,), jnp.int32),
        pltpu.VMEM((C,), jnp.int32),
        pltpu.VMEM((C,), jnp.float32),
        pltpu.VMEM((C,), jnp.float32),
        pltpu.VMEM((C,), jnp.float32),
        pltpu.VMEM((C,), jnp.float32),
        pltpu.VMEM((C,), jnp.float32),
        pltpu.SemaphoreType.DMA,
        pltpu.SemaphoreType.DMA,
        pltpu.SemaphoreType.DMA,
    ]

    @functools.partial(
        pl.kernel,
        out_type=jax.ShapeDtypeStruct((NC * N,), jnp.float32),
        mesh=_mesh,
        scratch_types=scratch,
    )
    def step(vin_hbm, edges_hbm, w_hbm, b_hbm, out_hbm,
             v_sh, acc_sh, stage, tmp_a, tmp_b,
             src0, src1, src2, dst0, dst1, dst2, w0, w1, w2,
             msg0, msg1, gsem, lsem, ssem):
        cid = lax.axis_index("c")
        sid = lax.axis_index("s")
        wid = sid * NC + cid
        off = jnp.minimum(sid * SLICE, N - SLICE)
        ebase = wid * EPW

        SRC = (src0, src1, src2)
        DST = (dst0, dst1, dst2)
        WB = (w0, w1, w2)
        MSG = (msg0, msg1)

        def issue_loads(g, r):
            b0 = ebase + g * C
            pltpu.async_copy(edges_hbm.at[pl.ds(b0, C)], SRC[r], lsem)
            pltpu.async_copy(edges_hbm.at[pl.ds(E + b0, C)], DST[r], lsem)
            pltpu.async_copy(w_hbm.at[pl.ds(b0, C)], WB[r], lsem)

        def wait_loads(g, r):
            b0 = ebase + g * C
            pltpu.make_async_copy(edges_hbm.at[pl.ds(b0, C)], SRC[r], lsem).wait()
            pltpu.make_async_copy(edges_hbm.at[pl.ds(E + b0, C)], DST[r], lsem).wait()
            pltpu.make_async_copy(w_hbm.at[pl.ds(b0, C)], WB[r], lsem).wait()

        def gather_mul(r, m):
            pltpu.async_copy(v_sh.at[SRC[r]], MSG[m], gsem).wait()
            wb, mb = WB[r], MSG[m]

            @plsc.parallel_loop(0, C // 16, unroll=10)
            def _mul(i):
                s = pl.ds(i * 16, 16)
                mb[s] = mb[s] * wb[s]

        def issue_scatter(r, m):
            pltpu.async_copy(MSG[m], acc_sh.at[DST[r]], ssem, add=True)

        def wait_scatter(r, m):
            pltpu.make_async_copy(MSG[m], acc_sh.at[DST[r]], ssem).wait()

        issue_loads(0, 0)
        issue_loads(1, 1)

        if first:
            pltpu.sync_copy(vin_hbm.at[pl.ds(off, SLICE)], stage)
        else:
            pltpu.sync_copy(vin_hbm.at[pl.ds(off, SLICE)], stage)
            pltpu.sync_copy(vin_hbm.at[pl.ds(N + off, SLICE)], tmp_a)
            pltpu.sync_copy(b_hbm.at[pl.ds(off, SLICE)], tmp_b)

            def combine_body(i, _):
                s = pl.ds(i * 16, 16)
                stage[s] = _tanh16(stage[s] + tmp_a[s] + tmp_b[s])
                return 0
            lax.fori_loop(0, SLICE // 16, combine_body, 0)

        def zero_body(i, _):
            tmp_a[pl.ds(i * 16, 16)] = jnp.zeros((16,), jnp.float32)
            return 0
        lax.fori_loop(0, SLICE // 16, zero_body, 0)

        pltpu.sync_copy(stage, v_sh.at[pl.ds(off, SLICE)])
        pltpu.sync_copy(tmp_a, acc_sh.at[pl.ds(off, SLICE)])
        plsc.subcore_barrier()

        def hex_body(h, _):
            g0 = 6 * h
            for j in range(6):
                wait_loads(g0 + j, j % 3)
                gather_mul(j % 3, j % 2)
                if j == 0:
                    @pl.when(h > 0)
                    def _():
                        wait_scatter(2, 1)
                else:
                    wait_scatter((j - 1) % 3, (j - 1) % 2)
                issue_scatter(j % 3, j % 2)
                issue_loads(g0 + j + 2, (j + 2) % 3)
            return 0
        lax.fori_loop(0, (NCHUNK - 2) // 6, hex_body, 0)
        wait_loads(NCHUNK - 2, (NCHUNK - 2) % 3)
        gather_mul((NCHUNK - 2) % 3, (NCHUNK - 2) % 2)
        wait_scatter((NCHUNK - 3) % 3, (NCHUNK - 3) % 2)
        issue_scatter((NCHUNK - 2) % 3, (NCHUNK - 2) % 2)
        wait_loads(NCHUNK - 1, (NCHUNK - 1) % 3)
        gather_mul((NCHUNK - 1) % 3, (NCHUNK - 1) % 2)
        wait_scatter((NCHUNK - 2) % 3, (NCHUNK - 2) % 2)
        issue_scatter((NCHUNK - 1) % 3, (NCHUNK - 1) % 2)
        wait_scatter((NCHUNK - 1) % 3, (NCHUNK - 1) % 2)
        plsc.subcore_barrier()

        pltpu.sync_copy(acc_sh.at[pl.ds(off, SLICE)], stage)
        pltpu.sync_copy(stage, out_hbm.at[pl.ds(cid * N + off, SLICE)])

    return step


_step_first = _make_step(first=True)
_step_next = _make_step(first=False)


@functools.partial(
    pl.kernel,
    out_type=jax.ShapeDtypeStruct((N,), jnp.float32),
    mesh=_mesh,
    scratch_types=[
        pltpu.VMEM((FS,), jnp.float32),
        pltpu.VMEM((FS,), jnp.float32),
        pltpu.VMEM((FS,), jnp.float32),
    ],
)
def _final(p_hbm, b_hbm, out_hbm, p0b, p1b, bb):
    cid = lax.axis_index("c")
    sid = lax.axis_index("s")
    wid = sid * NC + cid
    off = jnp.minimum(wid * FS, N - FS)
    pltpu.sync_copy(p_hbm.at[pl.ds(off, FS)], p0b)
    pltpu.sync_copy(p_hbm.at[pl.ds(N + off, FS)], p1b)
    pltpu.sync_copy(b_hbm.at[pl.ds(off, FS)], bb)

    def body(i, _):
        s = pl.ds(i * 16, 16)
        p0b[s] = _tanh16(p0b[s] + p1b[s] + bb[s])
        return 0
    lax.fori_loop(0, FS // 16, body, 0)
    pltpu.sync_copy(p0b, out_hbm.at[pl.ds(off, FS)])


def kernel(x, edge_index, synapse_weights, neuron_biases):
    edges = edge_index.reshape(-1)
    p = _step_first(x, edges, synapse_weights, neuron_biases)
    p = _step_next(p, edges, synapse_weights, neuron_biases)
    p = _step_next(p, edges, synapse_weights, neuron_biases)
    return _final(p, neuron_biases)


# speedup vs baseline: 389.7787x; 1.0053x over previous
import functools

import jax
import jax.numpy as jnp
from jax import lax
from jax.experimental import pallas as pl
from jax.experimental.pallas import tpu as pltpu
from jax.experimental.pallas import tpu_sc as plsc

N = 100000
E = 6400000
NC = 2
NS = 16
NW = NC * NS
EPW = E // NW
C = 4000
H = C // 2
NCHUNK = EPW // C
SLICE = 6272
FS = 3136

_mesh = plsc.VectorSubcoreMesh(core_axis_name="c", subcore_axis_name="s")


def _tanh16(t):
    e = jnp.exp(2.0 * t)
    return 1.0 - 2.0 / (e + 1.0)


def _make_step(first: bool):
    scratch = [
        pltpu.VMEM_SHARED((N,), jnp.float32),
        pltpu.VMEM_SHARED((N,), jnp.float32),
        pltpu.VMEM((SLICE,), jnp.float32),
        pltpu.VMEM((SLICE,), jnp.float32),
        pltpu.VMEM((SLICE,), jnp.float32),
        pltpu.VMEM((H,), jnp.int32),
        pltpu.VMEM((H,), jnp.int32),
        pltpu.VMEM((H,), jnp.int32),
        pltpu.VMEM((H,), jnp.int32),
        pltpu.VMEM((H,), jnp.int32),
        pltpu.VMEM((H,), jnp.int32),
        pltpu.VMEM((H,), jnp.int32),
        pltpu.VMEM((H,), jnp.int32),
        pltpu.VMEM((H,), jnp.int32),
        pltpu.VMEM((H,), jnp.int32),
        pltpu.VMEM((H,), jnp.int32),
        pltpu.VMEM((H,), jnp.int32),
        pltpu.VMEM((C,), jnp.float32),
        pltpu.VMEM((C,), jnp.float32),
        pltpu.VMEM((C,), jnp.float32),
        pltpu.VMEM((C,), jnp.float32),
        pltpu.VMEM((C,), jnp.float32),
        pltpu.SemaphoreType.DMA,
        pltpu.SemaphoreType.DMA,
        pltpu.SemaphoreType.DMA,
        pltpu.SemaphoreType.DMA,
    ]

    @functools.partial(
        pl.kernel,
        out_type=jax.ShapeDtypeStruct((NC * N,), jnp.float32),
        mesh=_mesh,
        scratch_types=scratch,
    )
    def step(vin_hbm, edges_hbm, w_hbm, b_hbm, out_hbm,
             v_sh, acc_sh, stage, tmp_a, tmp_b,
             sl0, sh0, sl1, sh1, sl2, sh2, dl0, dh0, dl1, dh1, dl2, dh2,
             w0, w1, w2, msg0, msg1, gsem, gsem2, lsem, ssem):
        cid = lax.axis_index("c")
        sid = lax.axis_index("s")
        wid = sid * NC + cid
        off = jnp.minimum(sid * SLICE, N - SLICE)
        ebase = wid * EPW

        SRCL, SRCH = (sl0, sl1, sl2), (sh0, sh1, sh2)
        DSTL, DSTH = (dl0, dl1, dl2), (dh0, dh1, dh2)
        WB = (w0, w1, w2)
        MSG = (msg0, msg1)

        def issue_loads(g, r):
            b0 = ebase + g * C
            pltpu.async_copy(edges_hbm.at[pl.ds(b0, H)], SRCL[r], lsem)
            pltpu.async_copy(edges_hbm.at[pl.ds(b0 + H, H)], SRCH[r], lsem)
            pltpu.async_copy(edges_hbm.at[pl.ds(E + b0, H)], DSTL[r], lsem)
            pltpu.async_copy(edges_hbm.at[pl.ds(E + b0 + H, H)], DSTH[r], lsem)
            pltpu.async_copy(w_hbm.at[pl.ds(b0, C)], WB[r], lsem)

        def wait_loads(g, r):
            b0 = ebase + g * C
            pltpu.make_async_copy(edges_hbm.at[pl.ds(b0, H)], SRCL[r], lsem).wait()
            pltpu.make_async_copy(edges_hbm.at[pl.ds(b0 + H, H)], SRCH[r], lsem).wait()
            pltpu.make_async_copy(edges_hbm.at[pl.ds(E + b0, H)], DSTL[r], lsem).wait()
            pltpu.make_async_copy(edges_hbm.at[pl.ds(E + b0 + H, H)], DSTH[r], lsem).wait()
            pltpu.make_async_copy(w_hbm.at[pl.ds(b0, C)], WB[r], lsem).wait()

        def gather_mul(r, m):
            wb, mb = WB[r], MSG[m]
            lo = pltpu.async_copy(v_sh.at[SRCL[r]], mb.at[pl.ds(0, H)], gsem)
            hi = pltpu.async_copy(v_sh.at[SRCH[r]], mb.at[pl.ds(H, H)], gsem2)
            lo.wait()

            @plsc.parallel_loop(0, H // 16, unroll=5)
            def _mul_lo(i):
                s = pl.ds(i * 16, 16)
                mb[s] = mb[s] * wb[s]

            hi.wait()

            @plsc.parallel_loop(0, H // 16, unroll=5)
            def _mul_hi(i):
                s = pl.ds(H + i * 16, 16)
                mb[s] = mb[s] * wb[s]

        def issue_scatter(r, m):
            mb = MSG[m]
            pltpu.async_copy(mb.at[pl.ds(0, H)], acc_sh.at[DSTL[r]], ssem, add=True)
            pltpu.async_copy(mb.at[pl.ds(H, H)], acc_sh.at[DSTH[r]], ssem, add=True)

        def wait_scatter(r, m):
            mb = MSG[m]
            pltpu.make_async_copy(mb.at[pl.ds(0, H)], acc_sh.at[DSTL[r]], ssem).wait()
            pltpu.make_async_copy(mb.at[pl.ds(H, H)], acc_sh.at[DSTH[r]], ssem).wait()

        issue_loads(0, 0)
        issue_loads(1, 1)

        if first:
            pltpu.sync_copy(vin_hbm.at[pl.ds(off, SLICE)], stage)
        else:
            pltpu.sync_copy(vin_hbm.at[pl.ds(off, SLICE)], stage)
            pltpu.sync_copy(vin_hbm.at[pl.ds(N + off, SLICE)], tmp_a)
            pltpu.sync_copy(b_hbm.at[pl.ds(off, SLICE)], tmp_b)

            def combine_body(i, _):
                s = pl.ds(i * 16, 16)
                stage[s] = _tanh16(stage[s] + tmp_a[s] + tmp_b[s])
                return 0
            lax.fori_loop(0, SLICE // 16, combine_body, 0)

        def zero_body(i, _):
            tmp_a[pl.ds(i * 16, 16)] = jnp.zeros((16,), jnp.float32)
            return 0
        lax.fori_loop(0, SLICE // 16, zero_body, 0)

        pltpu.sync_copy(stage, v_sh.at[pl.ds(off, SLICE)])
        pltpu.sync_copy(tmp_a, acc_sh.at[pl.ds(off, SLICE)])
        plsc.subcore_barrier()

        def hex_body(h, _):
            g0 = 6 * h
            for j in range(6):
                wait_loads(g0 + j, j % 3)
                gather_mul(j % 3, j % 2)
                if j == 0:
                    @pl.when(h > 0)
                    def _():
                        wait_scatter(2, 1)
                else:
                    wait_scatter((j - 1) % 3, (j - 1) % 2)
                issue_scatter(j % 3, j % 2)
                issue_loads(g0 + j + 2, (j + 2) % 3)
            return 0
        lax.fori_loop(0, (NCHUNK - 2) // 6, hex_body, 0)
        wait_loads(NCHUNK - 2, (NCHUNK - 2) % 3)
        gather_mul((NCHUNK - 2) % 3, (NCHUNK - 2) % 2)
        wait_scatter((NCHUNK - 3) % 3, (NCHUNK - 3) % 2)
        issue_scatter((NCHUNK - 2) % 3, (NCHUNK - 2) % 2)
        wait_loads(NCHUNK - 1, (NCHUNK - 1) % 3)
        gather_mul((NCHUNK - 1) % 3, (NCHUNK - 1) % 2)
        wait_scatter((NCHUNK - 2) % 3, (NCHUNK - 2) % 2)
        issue_scatter((NCHUNK - 1) % 3, (NCHUNK - 1) % 2)
        wait_scatter((NCHUNK - 1) % 3, (NCHUNK - 1) % 2)
        plsc.subcore_barrier()

        pltpu.sync_copy(acc_sh.at[pl.ds(off, SLICE)], stage)
        pltpu.sync_copy(stage, out_hbm.at[pl.ds(cid * N + off, SLICE)])

    return step


_step_first = _make_step(first=True)
_step_next = _make_step(first=False)


@functools.partial(
    pl.kernel,
    out_type=jax.ShapeDtypeStruct((N,), jnp.float32),
    mesh=_mesh,
    scratch_types=[
        pltpu.VMEM((FS,), jnp.float32),
        pltpu.VMEM((FS,), jnp.float32),
        pltpu.VMEM((FS,), jnp.float32),
    ],
)
def _final(p_hbm, b_hbm, out_hbm, p0b, p1b, bb):
    cid = lax.axis_index("c")
    sid = lax.axis_index("s")
    wid = sid * NC + cid
    off = jnp.minimum(wid * FS, N - FS)
    pltpu.sync_copy(p_hbm.at[pl.ds(off, FS)], p0b)
    pltpu.sync_copy(p_hbm.at[pl.ds(N + off, FS)], p1b)
    pltpu.sync_copy(b_hbm.at[pl.ds(off, FS)], bb)

    def body(i, _):
        s = pl.ds(i * 16, 16)
        p0b[s] = _tanh16(p0b[s] + p1b[s] + bb[s])
        return 0
    lax.fori_loop(0, FS // 16, body, 0)
    pltpu.sync_copy(p0b, out_hbm.at[pl.ds(off, FS)])


def kernel(x, edge_index, synapse_weights, neuron_biases):
    edges = edge_index.reshape(-1)
    p = _step_first(x, edges, synapse_weights, neuron_biases)
    p = _step_next(p, edges, synapse_weights, neuron_biases)
    p = _step_next(p, edges, synapse_weights, neuron_biases)
    return _final(p, neuron_biases)


# speedup vs baseline: 410.4242x; 1.0530x over previous
import functools

import jax
import jax.numpy as jnp
from jax import lax
from jax.experimental import pallas as pl
from jax.experimental.pallas import tpu as pltpu
from jax.experimental.pallas import tpu_sc as plsc

N = 100000
E = 6400000
NC = 2
NS = 16
NW = NC * NS
EPW = E // NW
C = 8000
H = C // 2
NCHUNK = EPW // C
SLICE = 6272
FS = 3136

_mesh = plsc.VectorSubcoreMesh(core_axis_name="c", subcore_axis_name="s")


def _tanh16(t):
    e = jnp.exp(2.0 * t)
    return 1.0 - 2.0 / (e + 1.0)


def _make_step(first: bool):
    scratch = [
        pltpu.VMEM_SHARED((N,), jnp.float32),
        pltpu.VMEM_SHARED((N,), jnp.float32),
        pltpu.VMEM((SLICE,), jnp.float32),
        pltpu.VMEM((SLICE,), jnp.float32),
        pltpu.VMEM((SLICE,), jnp.float32),
        pltpu.VMEM((H,), jnp.int32),
        pltpu.VMEM((H,), jnp.int32),
        pltpu.VMEM((H,), jnp.int32),
        pltpu.VMEM((H,), jnp.int32),
        pltpu.VMEM((H,), jnp.int32),
        pltpu.VMEM((H,), jnp.int32),
        pltpu.VMEM((H,), jnp.int32),
        pltpu.VMEM((H,), jnp.int32),
        pltpu.VMEM((H,), jnp.int32),
        pltpu.VMEM((H,), jnp.int32),
        pltpu.VMEM((H,), jnp.int32),
        pltpu.VMEM((H,), jnp.int32),
        pltpu.VMEM((C,), jnp.float32),
        pltpu.VMEM((C,), jnp.float32),
        pltpu.VMEM((C,), jnp.float32),
        pltpu.VMEM((C,), jnp.float32),
        pltpu.VMEM((C,), jnp.float32),
        pltpu.SemaphoreType.DMA,
        pltpu.SemaphoreType.DMA,
        pltpu.SemaphoreType.DMA,
        pltpu.SemaphoreType.DMA,
    ]

    @functools.partial(
        pl.kernel,
        out_type=jax.ShapeDtypeStruct((NC * N,), jnp.float32),
        mesh=_mesh,
        scratch_types=scratch,
    )
    def step(vin_hbm, edges_hbm, w_hbm, b_hbm, out_hbm,
             v_sh, acc_sh, stage, tmp_a, tmp_b,
             sl0, sh0, sl1, sh1, sl2, sh2, dl0, dh0, dl1, dh1, dl2, dh2,
             w0, w1, w2, msg0, msg1, gsem, gsem2, lsem, ssem):
        cid = lax.axis_index("c")
        sid = lax.axis_index("s")
        wid = sid * NC + cid
        off = jnp.minimum(sid * SLICE, N - SLICE)
        ebase = wid * EPW

        SRCL, SRCH = (sl0, sl1, sl2), (sh0, sh1, sh2)
        DSTL, DSTH = (dl0, dl1, dl2), (dh0, dh1, dh2)
        WB = (w0, w1, w2)
        MSG = (msg0, msg1)

        def issue_loads(g, r):
            b0 = ebase + g * C
            pltpu.async_copy(edges_hbm.at[pl.ds(b0, H)], SRCL[r], lsem)
            pltpu.async_copy(edges_hbm.at[pl.ds(b0 + H, H)], SRCH[r], lsem)
            pltpu.async_copy(edges_hbm.at[pl.ds(E + b0, H)], DSTL[r], lsem)
            pltpu.async_copy(edges_hbm.at[pl.ds(E + b0 + H, H)], DSTH[r], lsem)
            pltpu.async_copy(w_hbm.at[pl.ds(b0, C)], WB[r], lsem)

        def wait_loads(g, r):
            b0 = ebase + g * C
            pltpu.make_async_copy(edges_hbm.at[pl.ds(b0, H)], SRCL[r], lsem).wait()
            pltpu.make_async_copy(edges_hbm.at[pl.ds(b0 + H, H)], SRCH[r], lsem).wait()
            pltpu.make_async_copy(edges_hbm.at[pl.ds(E + b0, H)], DSTL[r], lsem).wait()
            pltpu.make_async_copy(edges_hbm.at[pl.ds(E + b0 + H, H)], DSTH[r], lsem).wait()
            pltpu.make_async_copy(w_hbm.at[pl.ds(b0, C)], WB[r], lsem).wait()

        def gather_mul(r, m):
            wb, mb = WB[r], MSG[m]
            lo = pltpu.async_copy(v_sh.at[SRCL[r]], mb.at[pl.ds(0, H)], gsem)
            hi = pltpu.async_copy(v_sh.at[SRCH[r]], mb.at[pl.ds(H, H)], gsem2)
            lo.wait()

            @plsc.parallel_loop(0, H // 16, unroll=5)
            def _mul_lo(i):
                s = pl.ds(i * 16, 16)
                mb[s] = mb[s] * wb[s]

            hi.wait()

            @plsc.parallel_loop(0, H // 16, unroll=5)
            def _mul_hi(i):
                s = pl.ds(H + i * 16, 16)
                mb[s] = mb[s] * wb[s]

        def issue_scatter(r, m):
            mb = MSG[m]
            pltpu.async_copy(mb.at[pl.ds(0, H)], acc_sh.at[DSTL[r]], ssem, add=True)
            pltpu.async_copy(mb.at[pl.ds(H, H)], acc_sh.at[DSTH[r]], ssem, add=True)

        def wait_scatter(r, m):
            mb = MSG[m]
            pltpu.make_async_copy(mb.at[pl.ds(0, H)], acc_sh.at[DSTL[r]], ssem).wait()
            pltpu.make_async_copy(mb.at[pl.ds(H, H)], acc_sh.at[DSTH[r]], ssem).wait()

        issue_loads(0, 0)
        issue_loads(1, 1)

        if first:
            pltpu.sync_copy(vin_hbm.at[pl.ds(off, SLICE)], stage)
        else:
            pltpu.sync_copy(vin_hbm.at[pl.ds(off, SLICE)], stage)
            pltpu.sync_copy(vin_hbm.at[pl.ds(N + off, SLICE)], tmp_a)
            pltpu.sync_copy(b_hbm.at[pl.ds(off, SLICE)], tmp_b)

            def combine_body(i, _):
                s = pl.ds(i * 16, 16)
                stage[s] = _tanh16(stage[s] + tmp_a[s] + tmp_b[s])
                return 0
            lax.fori_loop(0, SLICE // 16, combine_body, 0)

        def zero_body(i, _):
            tmp_a[pl.ds(i * 16, 16)] = jnp.zeros((16,), jnp.float32)
            return 0
        lax.fori_loop(0, SLICE // 16, zero_body, 0)

        pltpu.sync_copy(stage, v_sh.at[pl.ds(off, SLICE)])
        pltpu.sync_copy(tmp_a, acc_sh.at[pl.ds(off, SLICE)])
        plsc.subcore_barrier()

        def hex_body(h, _):
            g0 = 6 * h
            for j in range(6):
                wait_loads(g0 + j, j % 3)
                gather_mul(j % 3, j % 2)
                if j == 0:
                    @pl.when(h > 0)
                    def _():
                        wait_scatter(2, 1)
                else:
                    wait_scatter((j - 1) % 3, (j - 1) % 2)
                issue_scatter(j % 3, j % 2)
                issue_loads(g0 + j + 2, (j + 2) % 3)
            return 0
        NLOOP = 6 * ((NCHUNK - 2) // 6)
        lax.fori_loop(0, (NCHUNK - 2) // 6, hex_body, 0)

        def epi_chunk(g, pf=None):
            wait_loads(g, g % 3)
            gather_mul(g % 3, g % 2)
            wait_scatter((g - 1) % 3, (g - 1) % 2)
            issue_scatter(g % 3, g % 2)
            if pf is not None:
                issue_loads(pf, pf % 3)

        first_unprefetched = NLOOP + 2
        for g in range(NLOOP, NCHUNK):
            pf = first_unprefetched + (g - NLOOP)
            epi_chunk(g, pf if pf < NCHUNK else None)
        wait_scatter((NCHUNK - 1) % 3, (NCHUNK - 1) % 2)
        plsc.subcore_barrier()

        pltpu.sync_copy(acc_sh.at[pl.ds(off, SLICE)], stage)
        pltpu.sync_copy(stage, out_hbm.at[pl.ds(cid * N + off, SLICE)])

    return step


_step_first = _make_step(first=True)
_step_next = _make_step(first=False)


@functools.partial(
    pl.kernel,
    out_type=jax.ShapeDtypeStruct((N,), jnp.float32),
    mesh=_mesh,
    scratch_types=[
        pltpu.VMEM((FS,), jnp.float32),
        pltpu.VMEM((FS,), jnp.float32),
        pltpu.VMEM((FS,), jnp.float32),
    ],
)
def _final(p_hbm, b_hbm, out_hbm, p0b, p1b, bb):
    cid = lax.axis_index("c")
    sid = lax.axis_index("s")
    wid = sid * NC + cid
    off = jnp.minimum(wid * FS, N - FS)
    pltpu.sync_copy(p_hbm.at[pl.ds(off, FS)], p0b)
    pltpu.sync_copy(p_hbm.at[pl.ds(N + off, FS)], p1b)
    pltpu.sync_copy(b_hbm.at[pl.ds(off, FS)], bb)

    def body(i, _):
        s = pl.ds(i * 16, 16)
        p0b[s] = _tanh16(p0b[s] + p1b[s] + bb[s])
        return 0
    lax.fori_loop(0, FS // 16, body, 0)
    pltpu.sync_copy(p0b, out_hbm.at[pl.ds(off, FS)])


def kernel(x, edge_index, synapse_weights, neuron_biases):
    edges = edge_index.reshape(-1)
    p = _step_first(x, edges, synapse_weights, neuron_biases)
    p = _step_next(p, edges, synapse_weights, neuron_biases)
    p = _step_next(p, edges, synapse_weights, neuron_biases)
    return _final(p, neuron_biases)


# speedup vs baseline: 417.4032x; 1.0170x over previous
import functools

import jax
import jax.numpy as jnp
from jax import lax
from jax.experimental import pallas as pl
from jax.experimental.pallas import tpu as pltpu
from jax.experimental.pallas import tpu_sc as plsc

N = 100000
E = 6400000
NC = 2
NS = 16
NW = NC * NS
EPW = E // NW
C = 8000
H = C // 2
NCHUNK = EPW // C
SLICE = 6272
FS = 3136

_mesh = plsc.VectorSubcoreMesh(core_axis_name="c", subcore_axis_name="s")


def _tanh16(t):
    e = jnp.exp(2.0 * t)
    return 1.0 - 2.0 / (e + 1.0)


def _make_step(first: bool):
    scratch = [
        pltpu.VMEM_SHARED((N,), jnp.float32),
        pltpu.VMEM_SHARED((N,), jnp.float32),
        pltpu.VMEM((SLICE,), jnp.float32),
        pltpu.VMEM((SLICE,), jnp.float32),
        pltpu.VMEM((SLICE,), jnp.float32),
        pltpu.VMEM((H,), jnp.int32),
        pltpu.VMEM((H,), jnp.int32),
        pltpu.VMEM((H,), jnp.int32),
        pltpu.VMEM((H,), jnp.int32),
        pltpu.VMEM((H,), jnp.int32),
        pltpu.VMEM((H,), jnp.int32),
        pltpu.VMEM((H,), jnp.int32),
        pltpu.VMEM((H,), jnp.int32),
        pltpu.VMEM((H,), jnp.int32),
        pltpu.VMEM((H,), jnp.int32),
        pltpu.VMEM((H,), jnp.int32),
        pltpu.VMEM((H,), jnp.int32),
        pltpu.VMEM((C,), jnp.float32),
        pltpu.VMEM((C,), jnp.float32),
        pltpu.VMEM((C,), jnp.float32),
        pltpu.VMEM((C,), jnp.float32),
        pltpu.VMEM((C,), jnp.float32),
        pltpu.SemaphoreType.DMA,
        pltpu.SemaphoreType.DMA,
        pltpu.SemaphoreType.DMA,
        pltpu.SemaphoreType.DMA,
    ]

    @functools.partial(
        pl.kernel,
        out_type=jax.ShapeDtypeStruct((NC * N,), jnp.float32),
        mesh=_mesh,
        scratch_types=scratch,
    )
    def step(vin_hbm, edges_hbm, w_hbm, b_hbm, out_hbm,
             v_sh, acc_sh, stage, tmp_a, tmp_b,
             sl0, sh0, sl1, sh1, sl2, sh2, dl0, dh0, dl1, dh1, dl2, dh2,
             w0, w1, w2, msg0, msg1, gsem, gsem2, lsem, ssem):
        cid = lax.axis_index("c")
        sid = lax.axis_index("s")
        wid = sid * NC + cid
        off = jnp.minimum(sid * SLICE, N - SLICE)
        ebase = wid * EPW

        SRCL, SRCH = (sl0, sl1, sl2), (sh0, sh1, sh2)
        DSTL, DSTH = (dl0, dl1, dl2), (dh0, dh1, dh2)
        WB = (w0, w1, w2)
        MSG = (msg0, msg1)

        def issue_loads(g, r):
            b0 = ebase + g * C
            pltpu.async_copy(edges_hbm.at[pl.ds(b0, H)], SRCL[r], lsem)
            pltpu.async_copy(edges_hbm.at[pl.ds(b0 + H, H)], SRCH[r], lsem)
            pltpu.async_copy(edges_hbm.at[pl.ds(E + b0, H)], DSTL[r], lsem)
            pltpu.async_copy(edges_hbm.at[pl.ds(E + b0 + H, H)], DSTH[r], lsem)
            pltpu.async_copy(w_hbm.at[pl.ds(b0, C)], WB[r], lsem)

        def wait_loads(g, r):
            b0 = ebase + g * C
            pltpu.make_async_copy(edges_hbm.at[pl.ds(b0, H)], SRCL[r], lsem).wait()
            pltpu.make_async_copy(edges_hbm.at[pl.ds(b0 + H, H)], SRCH[r], lsem).wait()
            pltpu.make_async_copy(edges_hbm.at[pl.ds(E + b0, H)], DSTL[r], lsem).wait()
            pltpu.make_async_copy(edges_hbm.at[pl.ds(E + b0 + H, H)], DSTH[r], lsem).wait()
            pltpu.make_async_copy(w_hbm.at[pl.ds(b0, C)], WB[r], lsem).wait()

        def wait_scatter(r, m):
            mb = MSG[m]
            pltpu.make_async_copy(mb.at[pl.ds(0, H)], acc_sh.at[DSTL[r]], ssem).wait()
            pltpu.make_async_copy(mb.at[pl.ds(H, H)], acc_sh.at[DSTH[r]], ssem).wait()

        def process_chunk(r, m, pr, pm, wait_prev):
            wb, mb = WB[r], MSG[m]
            lo = pltpu.async_copy(v_sh.at[SRCL[r]], mb.at[pl.ds(0, H)], gsem)
            hi = pltpu.async_copy(v_sh.at[SRCH[r]], mb.at[pl.ds(H, H)], gsem2)
            lo.wait()

            @plsc.parallel_loop(0, H // 16, unroll=5)
            def _mul_lo(i):
                s = pl.ds(i * 16, 16)
                mb[s] = mb[s] * wb[s]

            if wait_prev is True:
                wait_scatter(pr, pm)
            elif wait_prev is not None:
                @pl.when(wait_prev)
                def _():
                    wait_scatter(pr, pm)
            pltpu.async_copy(mb.at[pl.ds(0, H)], acc_sh.at[DSTL[r]], ssem, add=True)
            hi.wait()

            @plsc.parallel_loop(0, H // 16, unroll=5)
            def _mul_hi(i):
                s = pl.ds(H + i * 16, 16)
                mb[s] = mb[s] * wb[s]

            pltpu.async_copy(mb.at[pl.ds(H, H)], acc_sh.at[DSTH[r]], ssem, add=True)

        issue_loads(0, 0)
        issue_loads(1, 1)

        if first:
            pltpu.sync_copy(vin_hbm.at[pl.ds(off, SLICE)], stage)
        else:
            pltpu.sync_copy(vin_hbm.at[pl.ds(off, SLICE)], stage)
            pltpu.sync_copy(vin_hbm.at[pl.ds(N + off, SLICE)], tmp_a)
            pltpu.sync_copy(b_hbm.at[pl.ds(off, SLICE)], tmp_b)

            def combine_body(i, _):
                s = pl.ds(i * 16, 16)
                stage[s] = _tanh16(stage[s] + tmp_a[s] + tmp_b[s])
                return 0
            lax.fori_loop(0, SLICE // 16, combine_body, 0)

        def zero_body(i, _):
            tmp_a[pl.ds(i * 16, 16)] = jnp.zeros((16,), jnp.float32)
            return 0
        lax.fori_loop(0, SLICE // 16, zero_body, 0)

        pltpu.sync_copy(stage, v_sh.at[pl.ds(off, SLICE)])
        pltpu.sync_copy(tmp_a, acc_sh.at[pl.ds(off, SLICE)])
        plsc.subcore_barrier()

        def hex_body(h, _):
            g0 = 6 * h
            for j in range(6):
                wait_loads(g0 + j, j % 3)
                process_chunk(j % 3, j % 2, (j - 1) % 3, (j - 1) % 2,
                              (h > 0) if j == 0 else True)
                issue_loads(g0 + j + 2, (j + 2) % 3)
            return 0
        NLOOP = 6 * ((NCHUNK - 2) // 6)
        lax.fori_loop(0, (NCHUNK - 2) // 6, hex_body, 0)

        def epi_chunk(g, pf=None):
            wait_loads(g, g % 3)
            process_chunk(g % 3, g % 2, (g - 1) % 3, (g - 1) % 2, True)
            if pf is not None:
                issue_loads(pf, pf % 3)

        first_unprefetched = NLOOP + 2
        for g in range(NLOOP, NCHUNK):
            pf = first_unprefetched + (g - NLOOP)
            epi_chunk(g, pf if pf < NCHUNK else None)
        wait_scatter((NCHUNK - 1) % 3, (NCHUNK - 1) % 2)
        plsc.subcore_barrier()

        pltpu.sync_copy(acc_sh.at[pl.ds(off, SLICE)], stage)
        pltpu.sync_copy(stage, out_hbm.at[pl.ds(cid * N + off, SLICE)])

    return step


_step_first = _make_step(first=True)
_step_next = _make_step(first=False)


@functools.partial(
    pl.kernel,
    out_type=jax.ShapeDtypeStruct((N,), jnp.float32),
    mesh=_mesh,
    scratch_types=[
        pltpu.VMEM((FS,), jnp.float32),
        pltpu.VMEM((FS,), jnp.float32),
        pltpu.VMEM((FS,), jnp.float32),
    ],
)
def _final(p_hbm, b_hbm, out_hbm, p0b, p1b, bb):
    cid = lax.axis_index("c")
    sid = lax.axis_index("s")
    wid = sid * NC + cid
    off = jnp.minimum(wid * FS, N - FS)
    pltpu.sync_copy(p_hbm.at[pl.ds(off, FS)], p0b)
    pltpu.sync_copy(p_hbm.at[pl.ds(N + off, FS)], p1b)
    pltpu.sync_copy(b_hbm.at[pl.ds(off, FS)], bb)

    def body(i, _):
        s = pl.ds(i * 16, 16)
        p0b[s] = _tanh16(p0b[s] + p1b[s] + bb[s])
        return 0
    lax.fori_loop(0, FS // 16, body, 0)
    pltpu.sync_copy(p0b, out_hbm.at[pl.ds(off, FS)])


def kernel(x, edge_index, synapse_weights, neuron_biases):
    edges = edge_index.reshape(-1)
    p = _step_first(x, edges, synapse_weights, neuron_biases)
    p = _step_next(p, edges, synapse_weights, neuron_biases)
    p = _step_next(p, edges, synapse_weights, neuron_biases)
    return _final(p, neuron_biases)


# speedup vs baseline: 439.2875x; 1.0524x over previous
import functools

import jax
import jax.numpy as jnp
from jax import lax
from jax.experimental import pallas as pl
from jax.experimental.pallas import tpu as pltpu
from jax.experimental.pallas import tpu_sc as plsc

N = 100000
E = 6400000
NC = 2
NS = 16
NW = NC * NS
EPW = E // NW
C = 8000
H = C // 2
NCHUNK = EPW // C
SLICE = 6272
FS = 3136

_mesh = plsc.VectorSubcoreMesh(core_axis_name="c", subcore_axis_name="s")


def _tanh16(t):
    e = jnp.exp(2.0 * t)
    return 1.0 - 2.0 / (e + 1.0)


def _make_step(first: bool):
    scratch = [
        pltpu.VMEM_SHARED((N,), jnp.float32),
        pltpu.VMEM_SHARED((N,), jnp.float32),
        pltpu.VMEM((SLICE,), jnp.float32),
        pltpu.VMEM((SLICE,), jnp.float32),
        pltpu.VMEM((SLICE,), jnp.float32),
        pltpu.VMEM((H,), jnp.int32),
        pltpu.VMEM((H,), jnp.int32),
        pltpu.VMEM((H,), jnp.int32),
        pltpu.VMEM((H,), jnp.int32),
        pltpu.VMEM((H,), jnp.int32),
        pltpu.VMEM((H,), jnp.int32),
        pltpu.VMEM((H,), jnp.int32),
        pltpu.VMEM((H,), jnp.int32),
        pltpu.VMEM((H,), jnp.int32),
        pltpu.VMEM((H,), jnp.int32),
        pltpu.VMEM((H,), jnp.int32),
        pltpu.VMEM((H,), jnp.int32),
        pltpu.VMEM((C,), jnp.float32),
        pltpu.VMEM((C,), jnp.float32),
        pltpu.VMEM((C,), jnp.float32),
        pltpu.VMEM((C,), jnp.float32),
        pltpu.VMEM((C,), jnp.float32),
        pltpu.SemaphoreType.DMA,
        pltpu.SemaphoreType.DMA,
        pltpu.SemaphoreType.DMA,
        pltpu.SemaphoreType.DMA,
    ]

    @functools.partial(
        pl.kernel,
        out_type=jax.ShapeDtypeStruct((NC * N,), jnp.float32),
        mesh=_mesh,
        scratch_types=scratch,
    )
    def step(vin_hbm, edges_hbm, w_hbm, b_hbm, out_hbm,
             v_sh, acc_sh, stage, tmp_a, tmp_b,
             sl0, sh0, sl1, sh1, sl2, sh2, dl0, dh0, dl1, dh1, dl2, dh2,
             w0, w1, w2, msg0, msg1, gsem, gsem2, lsem, ssem):
        cid = lax.axis_index("c")
        sid = lax.axis_index("s")
        wid = sid * NC + cid
        off = jnp.minimum(sid * SLICE, N - SLICE)
        ebase = wid * EPW

        SRCL, SRCH = (sl0, sl1, sl2), (sh0, sh1, sh2)
        DSTL, DSTH = (dl0, dl1, dl2), (dh0, dh1, dh2)
        WB = (w0, w1, w2)
        MSG = (msg0, msg1)

        def issue_loads(g, r):
            b0 = ebase + g * C
            pltpu.async_copy(edges_hbm.at[pl.ds(b0, H)], SRCL[r], lsem)
            pltpu.async_copy(edges_hbm.at[pl.ds(b0 + H, H)], SRCH[r], lsem)
            pltpu.async_copy(edges_hbm.at[pl.ds(E + b0, H)], DSTL[r], lsem)
            pltpu.async_copy(edges_hbm.at[pl.ds(E + b0 + H, H)], DSTH[r], lsem)
            pltpu.async_copy(w_hbm.at[pl.ds(b0, C)], WB[r], lsem)

        def wait_loads(g, r):
            b0 = ebase + g * C
            pltpu.make_async_copy(edges_hbm.at[pl.ds(b0, H)], SRCL[r], lsem).wait()
            pltpu.make_async_copy(edges_hbm.at[pl.ds(b0 + H, H)], SRCH[r], lsem).wait()
            pltpu.make_async_copy(edges_hbm.at[pl.ds(E + b0, H)], DSTL[r], lsem).wait()
            pltpu.make_async_copy(edges_hbm.at[pl.ds(E + b0 + H, H)], DSTH[r], lsem).wait()
            pltpu.make_async_copy(w_hbm.at[pl.ds(b0, C)], WB[r], lsem).wait()

        def wait_scatter(r, m):
            mb = MSG[m]
            pltpu.make_async_copy(mb.at[pl.ds(0, H)], acc_sh.at[DSTL[r]], ssem).wait()
            pltpu.make_async_copy(mb.at[pl.ds(H, H)], acc_sh.at[DSTH[r]], ssem).wait()

        def process_chunk(r, m, pr, pm, wait_prev):
            wb, mb = WB[r], MSG[m]
            lo = pltpu.async_copy(v_sh.at[SRCL[r]], mb.at[pl.ds(0, H)], gsem)
            hi = pltpu.async_copy(v_sh.at[SRCH[r]], mb.at[pl.ds(H, H)], gsem2)
            lo.wait()

            @plsc.parallel_loop(0, H // 16, unroll=5)
            def _mul_lo(i):
                s = pl.ds(i * 16, 16)
                mb[s] = mb[s] * wb[s]

            if wait_prev is True:
                wait_scatter(pr, pm)
            elif wait_prev is not None:
                @pl.when(wait_prev)
                def _():
                    wait_scatter(pr, pm)
            pltpu.async_copy(mb.at[pl.ds(0, H)], acc_sh.at[DSTL[r]], ssem, add=True)
            hi.wait()

            @plsc.parallel_loop(0, H // 16, unroll=5)
            def _mul_hi(i):
                s = pl.ds(H + i * 16, 16)
                mb[s] = mb[s] * wb[s]

            pltpu.async_copy(mb.at[pl.ds(H, H)], acc_sh.at[DSTH[r]], ssem, add=True)

        issue_loads(0, 0)
        issue_loads(1, 1)

        if first:
            a = pltpu.async_copy(vin_hbm.at[pl.ds(off, SLICE)], stage, gsem)
            a.wait()
        else:
            a = pltpu.async_copy(vin_hbm.at[pl.ds(off, SLICE)], stage, gsem)
            b = pltpu.async_copy(vin_hbm.at[pl.ds(N + off, SLICE)], tmp_a, gsem2)
            c = pltpu.async_copy(b_hbm.at[pl.ds(off, SLICE)], tmp_b, ssem)
            a.wait()
            b.wait()
            c.wait()

            @plsc.parallel_loop(0, SLICE // 16, unroll=8)
            def _combine(i):
                s = pl.ds(i * 16, 16)
                stage[s] = _tanh16(stage[s] + tmp_a[s] + tmp_b[s])

        @plsc.parallel_loop(0, SLICE // 16, unroll=8)
        def _zero(i):
            tmp_a[pl.ds(i * 16, 16)] = jnp.zeros((16,), jnp.float32)

        pv = pltpu.async_copy(stage, v_sh.at[pl.ds(off, SLICE)], gsem)
        pa = pltpu.async_copy(tmp_a, acc_sh.at[pl.ds(off, SLICE)], gsem2)
        pv.wait()
        pa.wait()
        plsc.subcore_barrier()

        def hex_body(h, _):
            g0 = 6 * h
            for j in range(6):
                wait_loads(g0 + j, j % 3)
                process_chunk(j % 3, j % 2, (j - 1) % 3, (j - 1) % 2,
                              (h > 0) if j == 0 else True)
                issue_loads(g0 + j + 2, (j + 2) % 3)
            return 0
        NLOOP = 6 * ((NCHUNK - 2) // 6)
        lax.fori_loop(0, (NCHUNK - 2) // 6, hex_body, 0)

        def epi_chunk(g, pf=None):
            wait_loads(g, g % 3)
            process_chunk(g % 3, g % 2, (g - 1) % 3, (g - 1) % 2, True)
            if pf is not None:
                issue_loads(pf, pf % 3)

        first_unprefetched = NLOOP + 2
        for g in range(NLOOP, NCHUNK):
            pf = first_unprefetched + (g - NLOOP)
            epi_chunk(g, pf if pf < NCHUNK else None)
        wait_scatter((NCHUNK - 1) % 3, (NCHUNK - 1) % 2)
        plsc.subcore_barrier()

        pltpu.sync_copy(acc_sh.at[pl.ds(off, SLICE)], stage)
        pltpu.sync_copy(stage, out_hbm.at[pl.ds(cid * N + off, SLICE)])

    return step


_step_first = _make_step(first=True)
_step_next = _make_step(first=False)


@functools.partial(
    pl.kernel,
    out_type=jax.ShapeDtypeStruct((N,), jnp.float32),
    mesh=_mesh,
    scratch_types=[
        pltpu.VMEM((FS,), jnp.float32),
        pltpu.VMEM((FS,), jnp.float32),
        pltpu.VMEM((FS,), jnp.float32),
    ],
)
def _final(p_hbm, b_hbm, out_hbm, p0b, p1b, bb):
    cid = lax.axis_index("c")
    sid = lax.axis_index("s")
    wid = sid * NC + cid
    off = jnp.minimum(wid * FS, N - FS)
    pltpu.sync_copy(p_hbm.at[pl.ds(off, FS)], p0b)
    pltpu.sync_copy(p_hbm.at[pl.ds(N + off, FS)], p1b)
    pltpu.sync_copy(b_hbm.at[pl.ds(off, FS)], bb)

    @plsc.parallel_loop(0, FS // 16, unroll=7)
    def _fin(i):
        s = pl.ds(i * 16, 16)
        p0b[s] = _tanh16(p0b[s] + p1b[s] + bb[s])
    pltpu.sync_copy(p0b, out_hbm.at[pl.ds(off, FS)])


def kernel(x, edge_index, synapse_weights, neuron_biases):
    edges = edge_index.reshape(-1)
    p = _step_first(x, edges, synapse_weights, neuron_biases)
    p = _step_next(p, edges, synapse_weights, neuron_biases)
    p = _step_next(p, edges, synapse_weights, neuron_biases)
    return _final(p, neuron_biases)
